# baseline (speedup 1.0000x reference)
import functools

import jax
import jax.numpy as jnp
from jax import lax
from jax.experimental import pallas as pl
from jax.experimental.pallas import tpu as pltpu
from jax.experimental.pallas import tpu_sc as plsc

C = 128
G = 20
GP = 32
N = 10000
NP = 10240
E = 320000
EP = 327680
HEADS = 2
DH = 64
QB = 128
NQ = NP // QB
KB = 512
NEG = -1e30

_NC, _NS = 2, 16
_NW = _NC * _NS
_EW = EP // _NW
_CH = 64
_NW_E = _NS
_EWX = EP // _NW_E
_NCH = _EWX // _CH
_RPT = NP // _NS


_NB = 4
_PH = 8
_CPP = _NCH // _PH


def _sc_segment_sum(h, src, dst, zeros):
    mesh = plsc.VectorSubcoreMesh(core_axis_name="c", subcore_axis_name="s")

    @functools.partial(
        pl.kernel,
        mesh=mesh,
        out_type=jax.ShapeDtypeStruct((_NC, NP, C), jnp.float32),
        scratch_types=[
            pltpu.VMEM((_CPP, _CH), jnp.int32),
            pltpu.VMEM((_CPP, _CH), jnp.int32),
            pltpu.VMEM((_NB, _CH, C), jnp.float32),
            pltpu.VMEM_SHARED((NP, C), jnp.float32),
        ] + [pltpu.SemaphoreType.DMA] * (2 * _NB),
    )
    def k(h_hbm, src_hbm, dst_hbm, z_hbm, out_hbm, sidx, didx, rows, acc,
          *sems):
        gs = sems[:_NB]
        ss = sems[_NB:]
        c = lax.axis_index("c")
        s = lax.axis_index("s")
        wid = s
        pltpu.sync_copy(z_hbm.at[pl.ds(s * _RPT, _RPT)],
                        acc.at[pl.ds(s * _RPT, _RPT)])
        plsc.subcore_barrier()

        @pl.when(c == 0)
        def _():
            for ph in range(_PH):
                pltpu.sync_copy(src_hbm.at[wid, pl.ds(ph * _CPP, _CPP)],
                                sidx)
                pltpu.sync_copy(dst_hbm.at[wid, pl.ds(ph * _CPP, _CPP)],
                                didx)
                for b in range(_NB):
                    pltpu.async_copy(h_hbm.at[sidx.at[b]], rows.at[b],
                                     gs[b])

                def body(j, carry):
                    for b in range(_NB):
                        cur = j * _NB + b
                        pltpu.make_async_copy(h_hbm.at[pl.ds(0, _CH)],
                                              rows.at[b], gs[b]).wait()
                        pltpu.sync_copy(rows.at[b], acc.at[didx.at[cur]],
                                        add=True)
                        nxt = cur + _NB

                        @pl.when(nxt < _CPP)
                        def _():
                            pltpu.async_copy(h_hbm.at[sidx.at[nxt]],
                                             rows.at[b], gs[b])
                    return carry

                lax.fori_loop(0, _CPP // _NB, body, 0)

        plsc.subcore_barrier()
        pltpu.sync_copy(acc.at[pl.ds(s * _RPT, _RPT)],
                        out_hbm.at[c, pl.ds(s * _RPT, _RPT)])

    return k(h, src, dst, zeros)


def _mm_t(a, w):
    return lax.dot_general(a, w, (((1,), (1,)), ((), ())),
                           preferred_element_type=jnp.float32)


def _bn(hm, maskf, g, b):
    hm = hm * maskf
    mu = jnp.sum(hm, axis=0, keepdims=True) * (1.0 / N)
    d = (hm - mu) * maskf
    var = jnp.sum(d * d, axis=0, keepdims=True) * (1.0 / N)
    return d * lax.rsqrt(var + 1e-5) * g + b


def _proj_body(x_ref, bcol_ref, w_ref, b_ref, out_ref):
    maskf = (bcol_ref[:] < G).astype(jnp.float32)
    out_ref[:] = (_mm_t(x_ref[:], w_ref[:]) + b_ref[:]) * maskf


def _proj(xp, bcol, w, b):
    return pl.pallas_call(
        _proj_body,
        out_shape=jax.ShapeDtypeStruct((NP, C), jnp.float32),
    )(xp, bcol, w, b)


def _ginsum_body(h_ref, agg_ref, out_ref):
    out_ref[:] = h_ref[:] + agg_ref[0] + agg_ref[1]


def _ginsum(h, agg):
    blk = 1024
    return pl.pallas_call(
        _ginsum_body,
        grid=(NP // blk,),
        in_specs=[
            pl.BlockSpec((blk, C), lambda i: (i, 0)),
            pl.BlockSpec((2, blk, C), lambda i: (0, i, 0)),
        ],
        out_specs=pl.BlockSpec((blk, C), lambda i: (i, 0)),
        out_shape=jax.ShapeDtypeStruct((NP, C), jnp.float32),
    )(h, agg)


def _layer_body(h_ref, gin_ref, bcol_ref, brow_ref, kbs_ref, kbn_ref,
                gw1_ref, gb1_ref, gw2_ref, gb2_ref,
                wq_ref, bq_ref, wk_ref, bk_ref, wv_ref, bv_ref,
                wo_ref, bo_ref,
                mw1_ref, mb1_ref, mw2_ref, mb2_ref,
                n1g_ref, n1b_ref, n2g_ref, n2b_ref, n3g_ref, n3b_ref,
                out_ref, qa_s, k_s, v_s):
    h = h_ref[:]
    maskf = (bcol_ref[:] < G).astype(jnp.float32)

    t = jnp.maximum(_mm_t(gin_ref[:], gw1_ref[:]) + gb1_ref[:], 0.0)
    local = _mm_t(t, gw2_ref[:]) + gb2_ref[:] + h
    out_ref[:] = _bn(local, maskf, n1g_ref[:], n1b_ref[:])

    qa_s[:] = _mm_t(h, wq_ref[:]) + bq_ref[:]
    k_s[:] = _mm_t(h, wk_ref[:]) + bk_ref[:]
    v_s[:] = _mm_t(h, wv_ref[:]) + bv_ref[:]

    def qstep(i, carry):
        off = i * QB
        bq = bcol_ref[pl.ds(off, QB), :]
        qblk = qa_s[pl.ds(off, QB), :]
        kb0 = kbs_ref[i]
        kbn = kbn_ref[i]

        def kstep(kb, carry):
            m0, l0, o0, m1, l1, o1 = carry
            koff = (kb0 + kb) * KB
            msk = bq == brow_ref[:, pl.ds(koff, KB)]
            outs = []
            for hd, (m, l, o) in ((0, (m0, l0, o0)), (1, (m1, l1, o1))):
                qh = qblk[:, hd * DH:(hd + 1) * DH]
                kh = k_s[pl.ds(koff, KB), pl.ds(hd * DH, DH)]
                sc = lax.dot_general(qh, kh, (((1,), (1,)), ((), ())),
                                     preferred_element_type=jnp.float32)
                sc = jnp.where(msk, sc * 0.125, NEG)
                mc = jnp.max(sc, axis=1, keepdims=True)
                mn = jnp.maximum(m, mc)
                p = jnp.exp(sc - mn)
                scale = jnp.exp(m - mn)
                ln = l * scale + jnp.sum(p, axis=1, keepdims=True)
                on = o * scale + lax.dot_general(
                    p, v_s[pl.ds(koff, KB), pl.ds(hd * DH, DH)],
                    (((1,), (0,)), ((), ())),
                    preferred_element_type=jnp.float32)
                outs.extend((mn, ln, on))
            return tuple(outs)

        mz = jnp.full((QB, 1), NEG, jnp.float32)
        lz = jnp.zeros((QB, 1), jnp.float32)
        oz = jnp.zeros((QB, DH), jnp.float32)
        m0, l0, o0, m1, l1, o1 = lax.fori_loop(
            0, kbn, kstep, (mz, lz, oz, mz, lz, oz))
        qa_s[pl.ds(off, QB), 0:DH] = o0 / l0
        qa_s[pl.ds(off, QB), DH:C] = o1 / l1
        return carry

    lax.fori_loop(0, NQ, qstep, 0)

    att = _mm_t(qa_s[:], wo_ref[:]) + bo_ref[:] + h
    att = _bn(att, maskf, n2g_ref[:], n2b_ref[:])

    out = out_ref[:] + att
    m1 = jnp.maximum(_mm_t(out, mw1_ref[:]) + mb1_ref[:], 0.0)
    out = out + _mm_t(m1, mw2_ref[:]) + mb2_ref[:]
    out = _bn(out, maskf, n3g_ref[:], n3b_ref[:])
    out_ref[:] = out * maskf


def _layer(h, gin, bcol, brow, kbs, kbn, lp):
    w_in = lp['attn_w_in']
    b_in = lp['attn_b_in']
    args = (
        h, gin, bcol, brow, kbs, kbn,
        lp['gin_w1'], lp['gin_b1'].reshape(1, C),
        lp['gin_w2'], lp['gin_b2'].reshape(1, C),
        w_in[0:C], b_in[0:C].reshape(1, C),
        w_in[C:2 * C], b_in[C:2 * C].reshape(1, C),
        w_in[2 * C:3 * C], b_in[2 * C:3 * C].reshape(1, C),
        lp['attn_w_out'], lp['attn_b_out'].reshape(1, C),
        lp['mlp_w1'], lp['mlp_b1'].reshape(1, 2 * C),
        lp['mlp_w2'], lp['mlp_b2'].reshape(1, C),
        lp['n1_g'].reshape(1, C), lp['n1_b'].reshape(1, C),
        lp['n2_g'].reshape(1, C), lp['n2_b'].reshape(1, C),
        lp['n3_g'].reshape(1, C), lp['n3_b'].reshape(1, C),
    )
    vspec = pl.BlockSpec(memory_space=pltpu.VMEM)
    sspec = pl.BlockSpec(memory_space=pltpu.SMEM)
    return pl.pallas_call(
        _layer_body,
        in_specs=[vspec, vspec, vspec, vspec, sspec, sspec]
        + [vspec] * 22,
        out_shape=jax.ShapeDtypeStruct((NP, C), jnp.float32),
        scratch_shapes=[
            pltpu.VMEM((NP, C), jnp.float32),
            pltpu.VMEM((NP, C), jnp.float32),
            pltpu.VMEM((NP, C), jnp.float32),
        ],
    )(*args)


def _head_body(h_ref, brow_ref, w1_ref, b1_ref, w2_ref, b2_ref,
               w3_ref, b3_ref, out_ref):
    oh = (lax.broadcasted_iota(jnp.int32, (GP, NP), 0)
          == brow_ref[:]).astype(jnp.float32)
    pooled = lax.dot_general(oh, h_ref[:], (((1,), (0,)), ((), ())),
                             preferred_element_type=jnp.float32)
    z = jnp.maximum(_mm_t(pooled, w1_ref[:]) + b1_ref[:], 0.0)
    z = jnp.maximum(_mm_t(z, w2_ref[:]) + b2_ref[:], 0.0)
    z = _mm_t(z, w3_ref[:]) + b3_ref[:]
    m = jnp.max(z, axis=1, keepdims=True)
    out_ref[:] = z - m - jnp.log(jnp.sum(jnp.exp(z - m), axis=1, keepdims=True))


def _head(h, brow, p):
    return pl.pallas_call(
        _head_body,
        out_shape=jax.ShapeDtypeStruct((GP, 10), jnp.float32),
    )(h, brow,
      p['head_w1'], p['head_b1'].reshape(1, C // 2),
      p['head_w2'], p['head_b2'].reshape(1, C // 4),
      p['head_w3'], p['head_b3'].reshape(1, 10))


def kernel(x, edge_index, batch, params):
    xp = jnp.pad(x, ((0, NP - N), (0, 0)))
    src = jnp.pad(edge_index[0], (0, EP - E)).reshape(_NW_E, _NCH, _CH)
    dst = jnp.pad(edge_index[1], (0, EP - E),
                  constant_values=NP - 1).reshape(_NW_E, _NCH, _CH)
    bp = jnp.pad(batch, (0, NP - N), constant_values=G)
    bcol = bp.reshape(NP, 1)
    brow = bp.reshape(1, NP)
    zeros = jnp.zeros((NP, C), jnp.float32)

    counts = jnp.bincount(bp, length=G + 1)
    ends = jnp.cumsum(counts)
    offs = ends - counts
    glo = bp[::QB]
    ghi = bp[QB - 1::QB]
    kbs = (offs[glo] // KB).astype(jnp.int32)
    kbn = ((ends[ghi] + KB - 1) // KB).astype(jnp.int32) - kbs

    h = _proj(xp, bcol, params['node_w'], params['node_b'].reshape(1, C))
    for lp in params['layers']:
        agg = _sc_segment_sum(h, src, dst, zeros)
        gin = _ginsum(h, agg)
        h = _layer(h, gin, bcol, brow, kbs, kbn, lp)
    out = _head(h, brow, params)
    return out[:G]


# speedup vs baseline: 1.0111x; 1.0111x over previous
import functools

import jax
import jax.numpy as jnp
from jax import lax
from jax.experimental import pallas as pl
from jax.experimental.pallas import tpu as pltpu
from jax.experimental.pallas import tpu_sc as plsc

C = 128
G = 20
GP = 32
N = 10000
NP = 10240
E = 320000
EP = 327680
HEADS = 2
DH = 64
QB = 128
NQ = NP // QB
KB = 512
NEG = -1e30

_NC, _NS = 2, 16
_NW = _NC * _NS
_EW = EP // _NW
_CH = 64
_NW_E = _NS
_EWX = EP // _NW_E
_NCH = _EWX // _CH
_RPT = NP // _NS


_NB = 4
_PH = 8
_CPP = _NCH // _PH


def _sc_segment_sum(h, src, dst, zeros):
    mesh = plsc.VectorSubcoreMesh(core_axis_name="c", subcore_axis_name="s")

    @functools.partial(
        pl.kernel,
        mesh=mesh,
        out_type=jax.ShapeDtypeStruct((_NC, NP, C), jnp.float32),
        scratch_types=[
            pltpu.VMEM((_CPP, _CH), jnp.int32),
            pltpu.VMEM((_CPP, _CH), jnp.int32),
            pltpu.VMEM((_NB, _CH, C), jnp.float32),
            pltpu.VMEM_SHARED((NP, C), jnp.float32),
        ] + [pltpu.SemaphoreType.DMA] * (2 * _NB),
    )
    def k(h_hbm, src_hbm, dst_hbm, z_hbm, out_hbm, sidx, didx, rows, acc,
          *sems):
        gs = sems[:_NB]
        ss = sems[_NB:]
        c = lax.axis_index("c")
        s = lax.axis_index("s")
        wid = s
        pltpu.sync_copy(z_hbm.at[pl.ds(s * _RPT, _RPT)],
                        acc.at[pl.ds(s * _RPT, _RPT)])
        plsc.subcore_barrier()

        @pl.when(c == 1)
        def _():
            for ph in range(_PH):
                pltpu.sync_copy(src_hbm.at[wid, pl.ds(ph * _CPP, _CPP)],
                                sidx)
                pltpu.sync_copy(dst_hbm.at[wid, pl.ds(ph * _CPP, _CPP)],
                                didx)
                for b in range(_NB):
                    pltpu.async_copy(h_hbm.at[sidx.at[b]], rows.at[b],
                                     gs[b])

                def body(j, carry):
                    for b in range(_NB):
                        cur = j * _NB + b
                        pltpu.make_async_copy(h_hbm.at[pl.ds(0, _CH)],
                                              rows.at[b], gs[b]).wait()
                        pltpu.sync_copy(rows.at[b], acc.at[didx.at[cur]],
                                        add=True)
                        nxt = cur + _NB

                        @pl.when(nxt < _CPP)
                        def _():
                            pltpu.async_copy(h_hbm.at[sidx.at[nxt]],
                                             rows.at[b], gs[b])
                    return carry

                lax.fori_loop(0, _CPP // _NB, body, 0)

        plsc.subcore_barrier()
        pltpu.sync_copy(acc.at[pl.ds(s * _RPT, _RPT)],
                        out_hbm.at[c, pl.ds(s * _RPT, _RPT)])

    return k(h, src, dst, zeros)


def _mm_t(a, w):
    return lax.dot_general(a, w, (((1,), (1,)), ((), ())),
                           preferred_element_type=jnp.float32)


def _bn(hm, maskf, g, b):
    hm = hm * maskf
    mu = jnp.sum(hm, axis=0, keepdims=True) * (1.0 / N)
    d = (hm - mu) * maskf
    var = jnp.sum(d * d, axis=0, keepdims=True) * (1.0 / N)
    return d * lax.rsqrt(var + 1e-5) * g + b


def _proj_body(x_ref, bcol_ref, w_ref, b_ref, out_ref):
    maskf = (bcol_ref[:] < G).astype(jnp.float32)
    out_ref[:] = (_mm_t(x_ref[:], w_ref[:]) + b_ref[:]) * maskf


def _proj(xp, bcol, w, b):
    return pl.pallas_call(
        _proj_body,
        out_shape=jax.ShapeDtypeStruct((NP, C), jnp.float32),
    )(xp, bcol, w, b)


def _ginsum_body(h_ref, agg_ref, out_ref):
    out_ref[:] = h_ref[:] + agg_ref[0] + agg_ref[1]


def _ginsum(h, agg):
    blk = 1024
    return pl.pallas_call(
        _ginsum_body,
        grid=(NP // blk,),
        in_specs=[
            pl.BlockSpec((blk, C), lambda i: (i, 0)),
            pl.BlockSpec((2, blk, C), lambda i: (0, i, 0)),
        ],
        out_specs=pl.BlockSpec((blk, C), lambda i: (i, 0)),
        out_shape=jax.ShapeDtypeStruct((NP, C), jnp.float32),
    )(h, agg)


def _layer_body(h_ref, gin_ref, bcol_ref, brow_ref, kbs_ref, kbn_ref,
                gw1_ref, gb1_ref, gw2_ref, gb2_ref,
                wq_ref, bq_ref, wk_ref, bk_ref, wv_ref, bv_ref,
                wo_ref, bo_ref,
                mw1_ref, mb1_ref, mw2_ref, mb2_ref,
                n1g_ref, n1b_ref, n2g_ref, n2b_ref, n3g_ref, n3b_ref,
                out_ref, qa_s, k_s, v_s):
    h = h_ref[:]
    maskf = (bcol_ref[:] < G).astype(jnp.float32)

    t = jnp.maximum(_mm_t(gin_ref[:], gw1_ref[:]) + gb1_ref[:], 0.0)
    local = _mm_t(t, gw2_ref[:]) + gb2_ref[:] + h
    out_ref[:] = _bn(local, maskf, n1g_ref[:], n1b_ref[:])

    qa_s[:] = _mm_t(h, wq_ref[:]) + bq_ref[:]
    k_s[:] = _mm_t(h, wk_ref[:]) + bk_ref[:]
    v_s[:] = _mm_t(h, wv_ref[:]) + bv_ref[:]

    def qstep(i, carry):
        off = i * QB
        bq = bcol_ref[pl.ds(off, QB), :]
        qblk = qa_s[pl.ds(off, QB), :]
        kb0 = kbs_ref[i]
        kbn = kbn_ref[i]

        def kstep(kb, carry):
            m0, l0, o0, m1, l1, o1 = carry
            koff = (kb0 + kb) * KB
            msk = bq == brow_ref[:, pl.ds(koff, KB)]
            outs = []
            for hd, (m, l, o) in ((0, (m0, l0, o0)), (1, (m1, l1, o1))):
                qh = qblk[:, hd * DH:(hd + 1) * DH]
                kh = k_s[pl.ds(koff, KB), pl.ds(hd * DH, DH)]
                sc = lax.dot_general(qh, kh, (((1,), (1,)), ((), ())),
                                     preferred_element_type=jnp.float32)
                sc = jnp.where(msk, sc * 0.125, NEG)
                mc = jnp.max(sc, axis=1, keepdims=True)
                mn = jnp.maximum(m, mc)
                p = jnp.exp(sc - mn)
                scale = jnp.exp(m - mn)
                ln = l * scale + jnp.sum(p, axis=1, keepdims=True)
                on = o * scale + lax.dot_general(
                    p, v_s[pl.ds(koff, KB), pl.ds(hd * DH, DH)],
                    (((1,), (0,)), ((), ())),
                    preferred_element_type=jnp.float32)
                outs.extend((mn, ln, on))
            return tuple(outs)

        mz = jnp.full((QB, 1), NEG, jnp.float32)
        lz = jnp.zeros((QB, 1), jnp.float32)
        oz = jnp.zeros((QB, DH), jnp.float32)
        m0, l0, o0, m1, l1, o1 = lax.fori_loop(
            0, kbn, kstep, (mz, lz, oz, mz, lz, oz))
        qa_s[pl.ds(off, QB), 0:DH] = o0 / l0
        qa_s[pl.ds(off, QB), DH:C] = o1 / l1
        return carry

    lax.fori_loop(0, NQ, qstep, 0)

    att = _mm_t(qa_s[:], wo_ref[:]) + bo_ref[:] + h
    att = _bn(att, maskf, n2g_ref[:], n2b_ref[:])

    out = out_ref[:] + att
    m1 = jnp.maximum(_mm_t(out, mw1_ref[:]) + mb1_ref[:], 0.0)
    out = out + _mm_t(m1, mw2_ref[:]) + mb2_ref[:]
    out = _bn(out, maskf, n3g_ref[:], n3b_ref[:])
    out_ref[:] = out * maskf


def _layer(h, gin, bcol, brow, kbs, kbn, lp):
    w_in = lp['attn_w_in']
    b_in = lp['attn_b_in']
    args = (
        h, gin, bcol, brow, kbs, kbn,
        lp['gin_w1'], lp['gin_b1'].reshape(1, C),
        lp['gin_w2'], lp['gin_b2'].reshape(1, C),
        w_in[0:C], b_in[0:C].reshape(1, C),
        w_in[C:2 * C], b_in[C:2 * C].reshape(1, C),
        w_in[2 * C:3 * C], b_in[2 * C:3 * C].reshape(1, C),
        lp['attn_w_out'], lp['attn_b_out'].reshape(1, C),
        lp['mlp_w1'], lp['mlp_b1'].reshape(1, 2 * C),
        lp['mlp_w2'], lp['mlp_b2'].reshape(1, C),
        lp['n1_g'].reshape(1, C), lp['n1_b'].reshape(1, C),
        lp['n2_g'].reshape(1, C), lp['n2_b'].reshape(1, C),
        lp['n3_g'].reshape(1, C), lp['n3_b'].reshape(1, C),
    )
    vspec = pl.BlockSpec(memory_space=pltpu.VMEM)
    sspec = pl.BlockSpec(memory_space=pltpu.SMEM)
    return pl.pallas_call(
        _layer_body,
        in_specs=[vspec, vspec, vspec, vspec, sspec, sspec]
        + [vspec] * 22,
        out_shape=jax.ShapeDtypeStruct((NP, C), jnp.float32),
        scratch_shapes=[
            pltpu.VMEM((NP, C), jnp.float32),
            pltpu.VMEM((NP, C), jnp.float32),
            pltpu.VMEM((NP, C), jnp.float32),
        ],
    )(*args)


def _head_body(h_ref, brow_ref, w1_ref, b1_ref, w2_ref, b2_ref,
               w3_ref, b3_ref, out_ref):
    oh = (lax.broadcasted_iota(jnp.int32, (GP, NP), 0)
          == brow_ref[:]).astype(jnp.float32)
    pooled = lax.dot_general(oh, h_ref[:], (((1,), (0,)), ((), ())),
                             preferred_element_type=jnp.float32)
    z = jnp.maximum(_mm_t(pooled, w1_ref[:]) + b1_ref[:], 0.0)
    z = jnp.maximum(_mm_t(z, w2_ref[:]) + b2_ref[:], 0.0)
    z = _mm_t(z, w3_ref[:]) + b3_ref[:]
    m = jnp.max(z, axis=1, keepdims=True)
    out_ref[:] = z - m - jnp.log(jnp.sum(jnp.exp(z - m), axis=1, keepdims=True))


def _head(h, brow, p):
    return pl.pallas_call(
        _head_body,
        out_shape=jax.ShapeDtypeStruct((GP, 10), jnp.float32),
    )(h, brow,
      p['head_w1'], p['head_b1'].reshape(1, C // 2),
      p['head_w2'], p['head_b2'].reshape(1, C // 4),
      p['head_w3'], p['head_b3'].reshape(1, 10))


def kernel(x, edge_index, batch, params):
    xp = jnp.pad(x, ((0, NP - N), (0, 0)))
    src = jnp.pad(edge_index[0], (0, EP - E)).reshape(_NW_E, _NCH, _CH)
    dst = jnp.pad(edge_index[1], (0, EP - E),
                  constant_values=NP - 1).reshape(_NW_E, _NCH, _CH)
    bp = jnp.pad(batch, (0, NP - N), constant_values=G)
    bcol = bp.reshape(NP, 1)
    brow = bp.reshape(1, NP)
    zeros = jnp.zeros((NP, C), jnp.float32)

    counts = jnp.bincount(bp, length=G + 1)
    ends = jnp.cumsum(counts)
    offs = ends - counts
    glo = bp[::QB]
    ghi = bp[QB - 1::QB]
    kbs = (offs[glo] // KB).astype(jnp.int32)
    kbn = ((ends[ghi] + KB - 1) // KB).astype(jnp.int32) - kbs

    h = _proj(xp, bcol, params['node_w'], params['node_b'].reshape(1, C))
    for lp in params['layers']:
        agg = _sc_segment_sum(h, src, dst, zeros)
        gin = _ginsum(h, agg)
        h = _layer(h, gin, bcol, brow, kbs, kbn, lp)
    out = _head(h, brow, params)
    return out[:G]


# speedup vs baseline: 1.4354x; 1.4197x over previous
import functools

import jax
import jax.numpy as jnp
from jax import lax
from jax.experimental import pallas as pl
from jax.experimental.pallas import tpu as pltpu
from jax.experimental.pallas import tpu_sc as plsc

C = 128
G = 20
GP = 32
N = 10000
NP = 10240
E = 320000
EP = 327680
HEADS = 2
DH = 64
QB = 128
NQ = NP // QB
KB = 512
NEG = -1e30

_NC, _NS = 2, 16
_NW = _NC * _NS
_EW = EP // _NW
_CH = 64
_NCH = _EW // _CH
_RPT = NP // _NS


_NB = 4
_PH = 4
_CPP = _NCH // _PH


def _sc_segment_sum(h, src, dst, zeros):
    mesh = plsc.VectorSubcoreMesh(core_axis_name="c", subcore_axis_name="s")

    @functools.partial(
        pl.kernel,
        mesh=mesh,
        out_type=jax.ShapeDtypeStruct((_NC, NP, C), jnp.float32),
        scratch_types=[
            pltpu.VMEM((_CPP, _CH), jnp.int32),
            pltpu.VMEM((_CPP, _CH), jnp.int32),
            pltpu.VMEM((_NB, _CH, C), jnp.float32),
            pltpu.VMEM_SHARED((NP, C), jnp.float32),
        ] + [pltpu.SemaphoreType.DMA] * (2 * _NB),
    )
    def k(h_hbm, src_hbm, dst_hbm, z_hbm, out_hbm, sidx, didx, rows, acc,
          *sems):
        gs = sems[:_NB]
        ss = sems[_NB:]
        c = lax.axis_index("c")
        s = lax.axis_index("s")
        wid = c * _NS + s
        pltpu.sync_copy(z_hbm.at[pl.ds(s * _RPT, _RPT)],
                        acc.at[pl.ds(s * _RPT, _RPT)])
        plsc.subcore_barrier()

        for ph in range(_PH):
            pltpu.sync_copy(src_hbm.at[wid, pl.ds(ph * _CPP, _CPP)], sidx)
            pltpu.sync_copy(dst_hbm.at[wid, pl.ds(ph * _CPP, _CPP)], didx)
            for b in range(_NB):
                pltpu.async_copy(h_hbm.at[sidx.at[b]], rows.at[b], gs[b])

            def body(j, carry):
                for b in range(_NB):
                    cur = j * _NB + b
                    pltpu.make_async_copy(h_hbm.at[pl.ds(0, _CH)],
                                          rows.at[b], gs[b]).wait()
                    pltpu.sync_copy(rows.at[b], acc.at[didx.at[cur]],
                                    add=True)
                    nxt = cur + _NB

                    @pl.when(nxt < _CPP)
                    def _():
                        pltpu.async_copy(h_hbm.at[sidx.at[nxt]], rows.at[b],
                                         gs[b])
                return carry

            lax.fori_loop(0, _CPP // _NB, body, 0)

        plsc.subcore_barrier()
        pltpu.sync_copy(acc.at[pl.ds(s * _RPT, _RPT)],
                        out_hbm.at[c, pl.ds(s * _RPT, _RPT)])

    return k(h, src, dst, zeros)


def _mm_t(a, w):
    return lax.dot_general(a, w, (((1,), (1,)), ((), ())),
                           preferred_element_type=jnp.float32)


def _bn(hm, maskf, g, b):
    hm = hm * maskf
    mu = jnp.sum(hm, axis=0, keepdims=True) * (1.0 / N)
    d = (hm - mu) * maskf
    var = jnp.sum(d * d, axis=0, keepdims=True) * (1.0 / N)
    return d * lax.rsqrt(var + 1e-5) * g + b


def _proj_body(x_ref, bcol_ref, w_ref, b_ref, out_ref):
    maskf = (bcol_ref[:] < G).astype(jnp.float32)
    out_ref[:] = (_mm_t(x_ref[:], w_ref[:]) + b_ref[:]) * maskf


def _proj(xp, bcol, w, b):
    return pl.pallas_call(
        _proj_body,
        out_shape=jax.ShapeDtypeStruct((NP, C), jnp.float32),
    )(xp, bcol, w, b)


def _attn_body(h_ref, bcol_ref, brow_ref, kbs_ref, kbn_ref,
               wq_ref, bq_ref, wk_ref, bk_ref, wv_ref, bv_ref,
               wo_ref, bo_ref, n2g_ref, n2b_ref,
               out_ref, qa_s, k_s, v_s):
    h = h_ref[:]
    maskf = (bcol_ref[:] < G).astype(jnp.float32)

    qa_s[:] = _mm_t(h, wq_ref[:]) + bq_ref[:]
    k_s[:] = _mm_t(h, wk_ref[:]) + bk_ref[:]
    v_s[:] = _mm_t(h, wv_ref[:]) + bv_ref[:]

    def qstep(i, carry):
        off = i * QB
        bq = bcol_ref[pl.ds(off, QB), :]
        qblk = qa_s[pl.ds(off, QB), :]
        kb0 = kbs_ref[i]
        kbn = kbn_ref[i]

        def kstep(kb, carry):
            m0, l0, o0, m1, l1, o1 = carry
            koff = (kb0 + kb) * KB
            msk = bq == brow_ref[:, pl.ds(koff, KB)]
            outs = []
            for hd, (m, l, o) in ((0, (m0, l0, o0)), (1, (m1, l1, o1))):
                qh = qblk[:, hd * DH:(hd + 1) * DH]
                kh = k_s[pl.ds(koff, KB), pl.ds(hd * DH, DH)]
                sc = lax.dot_general(qh, kh, (((1,), (1,)), ((), ())),
                                     preferred_element_type=jnp.float32)
                sc = jnp.where(msk, sc * 0.125, NEG)
                mc = jnp.max(sc, axis=1, keepdims=True)
                mn = jnp.maximum(m, mc)
                p = jnp.exp(sc - mn)
                scale = jnp.exp(m - mn)
                ln = l * scale + jnp.sum(p, axis=1, keepdims=True)
                on = o * scale + lax.dot_general(
                    p, v_s[pl.ds(koff, KB), pl.ds(hd * DH, DH)],
                    (((1,), (0,)), ((), ())),
                    preferred_element_type=jnp.float32)
                outs.extend((mn, ln, on))
            return tuple(outs)

        mz = jnp.full((QB, 1), NEG, jnp.float32)
        lz = jnp.zeros((QB, 1), jnp.float32)
        oz = jnp.zeros((QB, DH), jnp.float32)
        m0, l0, o0, m1, l1, o1 = lax.fori_loop(
            0, kbn, kstep, (mz, lz, oz, mz, lz, oz))
        qa_s[pl.ds(off, QB), 0:DH] = o0 / l0
        qa_s[pl.ds(off, QB), DH:C] = o1 / l1
        return carry

    lax.fori_loop(0, NQ, qstep, 0)

    att = _mm_t(qa_s[:], wo_ref[:]) + bo_ref[:] + h
    out_ref[:] = _bn(att, maskf, n2g_ref[:], n2b_ref[:])


def _attn(h, bcol, brow, kbs, kbn, lp):
    w_in = lp['attn_w_in']
    b_in = lp['attn_b_in']
    args = (
        h, bcol, brow, kbs, kbn,
        w_in[0:C], b_in[0:C].reshape(1, C),
        w_in[C:2 * C], b_in[C:2 * C].reshape(1, C),
        w_in[2 * C:3 * C], b_in[2 * C:3 * C].reshape(1, C),
        lp['attn_w_out'], lp['attn_b_out'].reshape(1, C),
        lp['n2_g'].reshape(1, C), lp['n2_b'].reshape(1, C),
    )
    vspec = pl.BlockSpec(memory_space=pltpu.VMEM)
    sspec = pl.BlockSpec(memory_space=pltpu.SMEM)
    return pl.pallas_call(
        _attn_body,
        in_specs=[vspec, vspec, vspec, sspec, sspec] + [vspec] * 10,
        out_shape=jax.ShapeDtypeStruct((NP, C), jnp.float32),
        scratch_shapes=[
            pltpu.VMEM((NP, C), jnp.float32),
            pltpu.VMEM((NP, C), jnp.float32),
            pltpu.VMEM((NP, C), jnp.float32),
        ],
    )(*args)


def _mix_body(h_ref, agg_ref, att_ref, bcol_ref,
              gw1_ref, gb1_ref, gw2_ref, gb2_ref,
              mw1_ref, mb1_ref, mw2_ref, mb2_ref,
              n1g_ref, n1b_ref, n3g_ref, n3b_ref, out_ref):
    h = h_ref[:]
    maskf = (bcol_ref[:] < G).astype(jnp.float32)
    gin = h + agg_ref[0] + agg_ref[1]
    t = jnp.maximum(_mm_t(gin, gw1_ref[:]) + gb1_ref[:], 0.0)
    local = _mm_t(t, gw2_ref[:]) + gb2_ref[:] + h
    local = _bn(local, maskf, n1g_ref[:], n1b_ref[:])
    out = local + att_ref[:]
    m1 = jnp.maximum(_mm_t(out, mw1_ref[:]) + mb1_ref[:], 0.0)
    out = out + _mm_t(m1, mw2_ref[:]) + mb2_ref[:]
    out = _bn(out, maskf, n3g_ref[:], n3b_ref[:])
    out_ref[:] = out * maskf


def _mix(h, agg, att, bcol, lp):
    return pl.pallas_call(
        _mix_body,
        out_shape=jax.ShapeDtypeStruct((NP, C), jnp.float32),
    )(h, agg, att, bcol,
      lp['gin_w1'], lp['gin_b1'].reshape(1, C),
      lp['gin_w2'], lp['gin_b2'].reshape(1, C),
      lp['mlp_w1'], lp['mlp_b1'].reshape(1, 2 * C),
      lp['mlp_w2'], lp['mlp_b2'].reshape(1, C),
      lp['n1_g'].reshape(1, C), lp['n1_b'].reshape(1, C),
      lp['n3_g'].reshape(1, C), lp['n3_b'].reshape(1, C))


def _head_body(h_ref, brow_ref, w1_ref, b1_ref, w2_ref, b2_ref,
               w3_ref, b3_ref, out_ref):
    oh = (lax.broadcasted_iota(jnp.int32, (GP, NP), 0)
          == brow_ref[:]).astype(jnp.float32)
    pooled = lax.dot_general(oh, h_ref[:], (((1,), (0,)), ((), ())),
                             preferred_element_type=jnp.float32)
    z = jnp.maximum(_mm_t(pooled, w1_ref[:]) + b1_ref[:], 0.0)
    z = jnp.maximum(_mm_t(z, w2_ref[:]) + b2_ref[:], 0.0)
    z = _mm_t(z, w3_ref[:]) + b3_ref[:]
    m = jnp.max(z, axis=1, keepdims=True)
    out_ref[:] = z - m - jnp.log(jnp.sum(jnp.exp(z - m), axis=1, keepdims=True))


def _head(h, brow, p):
    return pl.pallas_call(
        _head_body,
        out_shape=jax.ShapeDtypeStruct((GP, 10), jnp.float32),
    )(h, brow,
      p['head_w1'], p['head_b1'].reshape(1, C // 2),
      p['head_w2'], p['head_b2'].reshape(1, C // 4),
      p['head_w3'], p['head_b3'].reshape(1, 10))


def kernel(x, edge_index, batch, params):
    xp = jnp.pad(x, ((0, NP - N), (0, 0)))
    src = jnp.pad(edge_index[0], (0, EP - E)).reshape(_NW, _NCH, _CH)
    dst = jnp.pad(edge_index[1], (0, EP - E),
                  constant_values=NP - 1).reshape(_NW, _NCH, _CH)
    bp = jnp.pad(batch, (0, NP - N), constant_values=G)
    bcol = bp.reshape(NP, 1)
    brow = bp.reshape(1, NP)
    zeros = jnp.zeros((NP, C), jnp.float32)

    counts = jnp.bincount(bp, length=G + 1)
    ends = jnp.cumsum(counts)
    offs = ends - counts
    glo = bp[::QB]
    ghi = bp[QB - 1::QB]
    kbs = (offs[glo] // KB).astype(jnp.int32)
    kbn = ((ends[ghi] + KB - 1) // KB).astype(jnp.int32) - kbs

    h = _proj(xp, bcol, params['node_w'], params['node_b'].reshape(1, C))
    for lp in params['layers']:
        agg = _sc_segment_sum(h, src, dst, zeros)
        att = _attn(h, bcol, brow, kbs, kbn, lp)
        h = _mix(h, agg, att, bcol, lp)
    out = _head(h, brow, params)
    return out[:G]


# speedup vs baseline: 1.4439x; 1.0059x over previous
import functools

import jax
import jax.numpy as jnp
from jax import lax
from jax.experimental import pallas as pl
from jax.experimental.pallas import tpu as pltpu
from jax.experimental.pallas import tpu_sc as plsc

C = 128
G = 20
GP = 32
N = 10000
NP = 10240
E = 320000
EP = 327680
HEADS = 2
DH = 64
QB = 128
NQ = NP // QB
KB = 512
NEG = -1e30

_NC, _NS = 2, 16
_NW = _NC * _NS
_EW = EP // _NW
_CH = 64
_NCH = _EW // _CH
_RPT = NP // _NS


_NB = 4
_PH = 4
_CPP = _NCH // _PH


def _sc_segment_sum(h, src, dst, zeros):
    mesh = plsc.VectorSubcoreMesh(core_axis_name="c", subcore_axis_name="s")

    @functools.partial(
        pl.kernel,
        mesh=mesh,
        out_type=jax.ShapeDtypeStruct((_NC, NP, C), jnp.float32),
        scratch_types=[
            pltpu.VMEM((_CPP, _CH), jnp.int32),
            pltpu.VMEM((_CPP, _CH), jnp.int32),
            pltpu.VMEM((_NB, _CH, C), jnp.float32),
            pltpu.VMEM_SHARED((NP, C), jnp.float32),
        ] + [pltpu.SemaphoreType.DMA] * (2 * _NB),
    )
    def k(h_hbm, src_hbm, dst_hbm, z_hbm, out_hbm, sidx, didx, rows, acc,
          *sems):
        gs = sems[:_NB]
        ss = sems[_NB:]
        c = lax.axis_index("c")
        s = lax.axis_index("s")
        wid = c * _NS + s
        pltpu.sync_copy(z_hbm.at[pl.ds(s * _RPT, _RPT)],
                        acc.at[pl.ds(s * _RPT, _RPT)])
        plsc.subcore_barrier()

        for ph in range(_PH):
            pltpu.sync_copy(src_hbm.at[wid, pl.ds(ph * _CPP, _CPP)], sidx)
            pltpu.sync_copy(dst_hbm.at[wid, pl.ds(ph * _CPP, _CPP)], didx)
            for b in range(_NB):
                pltpu.async_copy(h_hbm.at[sidx.at[b]], rows.at[b], gs[b])

            def body(j, carry):
                for b in range(_NB):
                    cur = j * _NB + b
                    pltpu.make_async_copy(h_hbm.at[pl.ds(0, _CH)],
                                          rows.at[b], gs[b]).wait()
                    pltpu.sync_copy(rows.at[b], acc.at[didx.at[cur]],
                                    add=True)
                    nxt = cur + _NB

                    @pl.when(nxt < _CPP)
                    def _():
                        pltpu.async_copy(h_hbm.at[sidx.at[nxt]], rows.at[b],
                                         gs[b])
                return carry

            lax.fori_loop(0, _CPP // _NB, body, 0)

        plsc.subcore_barrier()
        pltpu.sync_copy(acc.at[pl.ds(s * _RPT, _RPT)],
                        out_hbm.at[c, pl.ds(s * _RPT, _RPT)])

    return k(h, src, dst, zeros)


def _mm_t(a, w):
    return lax.dot_general(a, w, (((1,), (1,)), ((), ())),
                           preferred_element_type=jnp.float32)


def _bn(hm, maskf, g, b):
    hm = hm * maskf
    mu = jnp.sum(hm, axis=0, keepdims=True) * (1.0 / N)
    d = (hm - mu) * maskf
    var = jnp.sum(d * d, axis=0, keepdims=True) * (1.0 / N)
    return d * lax.rsqrt(var + 1e-5) * g + b


def _proj_body(x_ref, bcol_ref, w_ref, b_ref, out_ref):
    maskf = (bcol_ref[:] < G).astype(jnp.float32)
    out_ref[:] = (_mm_t(x_ref[:], w_ref[:]) + b_ref[:]) * maskf


def _proj(xp, bcol, w, b):
    return pl.pallas_call(
        _proj_body,
        out_shape=jax.ShapeDtypeStruct((NP, C), jnp.float32),
    )(xp, bcol, w, b)


def _attn_body(h_ref, bcol_ref, brow_ref, kbs_ref, kbn_ref,
               wq_ref, bq_ref, wk_ref, bk_ref, wv_ref, bv_ref,
               wo_ref, bo_ref, n2g_ref, n2b_ref,
               out_ref, qa_s, k_s, v_s):
    h = h_ref[:]
    maskf = (bcol_ref[:] < G).astype(jnp.float32)

    qa_s[:] = _mm_t(h, wq_ref[:]) + bq_ref[:]
    k_s[:] = _mm_t(h, wk_ref[:]) + bk_ref[:]
    v_s[:] = _mm_t(h, wv_ref[:]) + bv_ref[:]

    def qstep(i, carry):
        off = i * QB
        bq = bcol_ref[pl.ds(off, QB), :]
        qblk = qa_s[pl.ds(off, QB), :]
        kb0 = kbs_ref[i]
        kbn = kbn_ref[i]

        def kstep(kb, carry):
            m0, l0, o0, m1, l1, o1 = carry
            koff = (kb0 + kb) * KB
            msk = bq == brow_ref[:, pl.ds(koff, KB)]
            outs = []
            for hd, (m, l, o) in ((0, (m0, l0, o0)), (1, (m1, l1, o1))):
                qh = qblk[:, hd * DH:(hd + 1) * DH]
                kh = k_s[pl.ds(koff, KB), pl.ds(hd * DH, DH)]
                sc = lax.dot_general(qh, kh, (((1,), (1,)), ((), ())),
                                     preferred_element_type=jnp.float32)
                sc = jnp.where(msk, sc * 0.125, NEG)
                mc = jnp.max(sc, axis=1, keepdims=True)
                mn = jnp.maximum(m, mc)
                p = jnp.exp(sc - mn)
                scale = jnp.exp(m - mn)
                ln = l * scale + jnp.sum(p, axis=1, keepdims=True)
                on = o * scale + lax.dot_general(
                    p, v_s[pl.ds(koff, KB), pl.ds(hd * DH, DH)],
                    (((1,), (0,)), ((), ())),
                    preferred_element_type=jnp.float32)
                outs.extend((mn, ln, on))
            return tuple(outs)

        mz = jnp.full((QB, 1), NEG, jnp.float32)
        lz = jnp.zeros((QB, 1), jnp.float32)
        oz = jnp.zeros((QB, DH), jnp.float32)
        m0, l0, o0, m1, l1, o1 = lax.fori_loop(
            0, kbn, kstep, (mz, lz, oz, mz, lz, oz))
        qa_s[pl.ds(off, QB), 0:DH] = o0 / l0
        qa_s[pl.ds(off, QB), DH:C] = o1 / l1
        return carry

    lax.fori_loop(0, NQ, qstep, 0)

    att = _mm_t(qa_s[:], wo_ref[:]) + bo_ref[:] + h
    out_ref[:] = _bn(att, maskf, n2g_ref[:], n2b_ref[:])


def _attn(h, bcol, brow, kbs, kbn, lp):
    w_in = lp['attn_w_in']
    b_in = lp['attn_b_in']
    args = (
        h, bcol, brow, kbs, kbn,
        w_in[0:C], b_in[0:C].reshape(1, C),
        w_in[C:2 * C], b_in[C:2 * C].reshape(1, C),
        w_in[2 * C:3 * C], b_in[2 * C:3 * C].reshape(1, C),
        lp['attn_w_out'], lp['attn_b_out'].reshape(1, C),
        lp['n2_g'].reshape(1, C), lp['n2_b'].reshape(1, C),
    )
    vspec = pl.BlockSpec(memory_space=pltpu.VMEM)
    sspec = pl.BlockSpec(memory_space=pltpu.SMEM)
    return pl.pallas_call(
        _attn_body,
        in_specs=[vspec, vspec, vspec, sspec, sspec] + [vspec] * 10,
        out_shape=jax.ShapeDtypeStruct((NP, C), jnp.float32),
        scratch_shapes=[
            pltpu.VMEM((NP, C), jnp.float32),
            pltpu.VMEM((NP, C), jnp.float32),
            pltpu.VMEM((NP, C), jnp.float32),
        ],
    )(*args)


def _mix_core(h_ref, agg_ref, att_ref, bcol_ref,
              gw1_ref, gb1_ref, gw2_ref, gb2_ref,
              mw1_ref, mb1_ref, mw2_ref, mb2_ref,
              n1g_ref, n1b_ref, n3g_ref, n3b_ref):
    h = h_ref[:]
    maskf = (bcol_ref[:] < G).astype(jnp.float32)
    gin = h + agg_ref[0] + agg_ref[1]
    t = jnp.maximum(_mm_t(gin, gw1_ref[:]) + gb1_ref[:], 0.0)
    local = _mm_t(t, gw2_ref[:]) + gb2_ref[:] + h
    local = _bn(local, maskf, n1g_ref[:], n1b_ref[:])
    out = local + att_ref[:]
    m1 = jnp.maximum(_mm_t(out, mw1_ref[:]) + mb1_ref[:], 0.0)
    out = out + _mm_t(m1, mw2_ref[:]) + mb2_ref[:]
    out = _bn(out, maskf, n3g_ref[:], n3b_ref[:])
    return out * maskf


def _mix_body(*refs):
    out_ref = refs[-1]
    out_ref[:] = _mix_core(*refs[:-1])


def _mix_args(h, agg, att, bcol, lp):
    return (h, agg, att, bcol,
            lp['gin_w1'], lp['gin_b1'].reshape(1, C),
            lp['gin_w2'], lp['gin_b2'].reshape(1, C),
            lp['mlp_w1'], lp['mlp_b1'].reshape(1, 2 * C),
            lp['mlp_w2'], lp['mlp_b2'].reshape(1, C),
            lp['n1_g'].reshape(1, C), lp['n1_b'].reshape(1, C),
            lp['n3_g'].reshape(1, C), lp['n3_b'].reshape(1, C))


def _mix(h, agg, att, bcol, lp):
    return pl.pallas_call(
        _mix_body,
        out_shape=jax.ShapeDtypeStruct((NP, C), jnp.float32),
    )(*_mix_args(h, agg, att, bcol, lp))


def _mix_head_body(*refs):
    (brow_ref, w1_ref, b1_ref, w2_ref, b2_ref, w3_ref, b3_ref,
     out_ref) = refs[-8:]
    out = _mix_core(*refs[:-8])
    oh = (lax.broadcasted_iota(jnp.int32, (GP, NP), 0)
          == brow_ref[:]).astype(jnp.float32)
    pooled = lax.dot_general(oh, out, (((1,), (0,)), ((), ())),
                             preferred_element_type=jnp.float32)
    z = jnp.maximum(_mm_t(pooled, w1_ref[:]) + b1_ref[:], 0.0)
    z = jnp.maximum(_mm_t(z, w2_ref[:]) + b2_ref[:], 0.0)
    z = _mm_t(z, w3_ref[:]) + b3_ref[:]
    m = jnp.max(z, axis=1, keepdims=True)
    out_ref[:] = z - m - jnp.log(jnp.sum(jnp.exp(z - m), axis=1,
                                         keepdims=True))


def _mix_head(h, agg, att, bcol, brow, lp, p):
    return pl.pallas_call(
        _mix_head_body,
        out_shape=jax.ShapeDtypeStruct((GP, 10), jnp.float32),
    )(*_mix_args(h, agg, att, bcol, lp), brow,
      p['head_w1'], p['head_b1'].reshape(1, C // 2),
      p['head_w2'], p['head_b2'].reshape(1, C // 4),
      p['head_w3'], p['head_b3'].reshape(1, 10))


def kernel(x, edge_index, batch, params):
    xp = jnp.pad(x, ((0, NP - N), (0, 0)))
    src = jnp.pad(edge_index[0], (0, EP - E)).reshape(_NW, _NCH, _CH)
    dst = jnp.pad(edge_index[1], (0, EP - E),
                  constant_values=NP - 1).reshape(_NW, _NCH, _CH)
    bp = jnp.pad(batch, (0, NP - N), constant_values=G)
    bcol = bp.reshape(NP, 1)
    brow = bp.reshape(1, NP)
    zeros = jnp.zeros((NP, C), jnp.float32)

    counts = jnp.bincount(bp, length=G + 1)
    ends = jnp.cumsum(counts)
    offs = ends - counts
    glo = bp[::QB]
    ghi = bp[QB - 1::QB]
    kbs = (offs[glo] // KB).astype(jnp.int32)
    kbn = ((ends[ghi] + KB - 1) // KB).astype(jnp.int32) - kbs

    h = _proj(xp, bcol, params['node_w'], params['node_b'].reshape(1, C))
    layers = params['layers']
    for lp in layers[:-1]:
        agg = _sc_segment_sum(h, src, dst, zeros)
        att = _attn(h, bcol, brow, kbs, kbn, lp)
        h = _mix(h, agg, att, bcol, lp)
    lp = layers[-1]
    agg = _sc_segment_sum(h, src, dst, zeros)
    att = _attn(h, bcol, brow, kbs, kbn, lp)
    out = _mix_head(h, agg, att, bcol, brow, lp, params)
    return out[:G]
